# baseline (speedup 1.0000x reference)
import functools

import jax
import jax.numpy as jnp
from jax import lax
from jax.experimental import pallas as pl
from jax.experimental.pallas import tpu as pltpu
from jax.experimental.pallas import tpu_sc as plsc

BB, HH, WW = 64, 64, 64
VV = BB * HH * WW
NACT = 50000
EPSV = 1e-5

_NC, _NS = 2, 16
_NW = _NC * _NS
_NCH = 13
_WCHUNK = _NCH * 128
_NPAD = _NW * _WCHUNK
_ZROWS = VV // _NS


def _sc_scatter(idxe, upd, zeros_words):
    mesh = plsc.VectorSubcoreMesh(core_axis_name="c", subcore_axis_name="s")
    zwords = (VV * 4) // _NS

    @functools.partial(
        pl.kernel,
        mesh=mesh,
        compiler_params=pltpu.CompilerParams(use_tc_tiling_on_sc=False),
        out_type=jax.ShapeDtypeStruct((_NC * VV * 4,), jnp.float32),
        scratch_types=[
            pltpu.VMEM((4 * _NCH, 128), jnp.int32),
            pltpu.VMEM((4 * _WCHUNK,), jnp.float32),
            pltpu.VMEM_SHARED((VV * 4,), jnp.float32),
        ],
    )
    def k(idx_hbm, upd_hbm, zero_hbm, out_hbm, idx_v, upd_v, canvas):
        cid = lax.axis_index("c")
        sid = lax.axis_index("s")
        wid = cid * _NS + sid
        w0 = pl.multiple_of(sid * zwords, zwords)
        pltpu.sync_copy(zero_hbm, canvas.at[pl.ds(w0, zwords)])
        pltpu.sync_copy(idx_hbm.at[wid], idx_v)
        pltpu.sync_copy(upd_hbm.at[wid], upd_v)
        plsc.subcore_barrier()
        def body(j, carry):
            off = pl.multiple_of(j * 128, 128)
            pltpu.sync_copy(upd_v.at[pl.ds(off, 128)],
                            canvas.at[idx_v.at[j]], add=True)
            return carry

        lax.fori_loop(0, 4 * _NCH, body, 0)
        plsc.subcore_barrier()
        out0 = pl.multiple_of(cid * VV * 4 + sid * zwords, zwords)
        pltpu.sync_copy(canvas.at[pl.ds(w0, zwords)],
                        out_hbm.at[pl.ds(out0, zwords)])

    return k(idxe, upd, zeros_words)


def _sc_gather(table16, idx3):
    mesh = plsc.VectorSubcoreMesh(core_axis_name="c", subcore_axis_name="s")

    @functools.partial(
        pl.kernel,
        mesh=mesh,
        compiler_params=pltpu.CompilerParams(use_tc_tiling_on_sc=False),
        out_type=jax.ShapeDtypeStruct((_NW, _WCHUNK, 16), jnp.float32),
        scratch_types=[
            pltpu.VMEM((_NCH, 128), jnp.int32),
            pltpu.VMEM((_WCHUNK, 16), jnp.float32),
        ],
    )
    def k(tab_hbm, idx_hbm, out_hbm, idx_v, rows_v):
        cid = lax.axis_index("c")
        sid = lax.axis_index("s")
        wid = cid * _NS + sid
        pltpu.sync_copy(idx_hbm.at[wid], idx_v)
        for j in range(_NCH):
            pltpu.sync_copy(tab_hbm.at[idx_v.at[j]],
                            rows_v.at[pl.ds(j * 128, 128)])
        pltpu.sync_copy(rows_v, out_hbm.at[wid])

    return k(table16, idx3)


def _conv9(xp, w):
    h = xp.shape[0] - 2
    wd = xp.shape[1] - 2
    cin = xp.shape[2]
    cout = w.shape[3]
    acc = jnp.zeros((h * wd, cout), jnp.float32)
    for ky in range(3):
        for kx in range(3):
            patch = xp[ky:ky + h, kx:kx + wd, :].reshape(h * wd, cin)
            acc += jnp.dot(patch, w[ky, kx], preferred_element_type=jnp.float32)
    return acc.reshape(h, wd, cout)


def _pool_down(m):
    h = m.shape[0]
    mp = jnp.pad(m, ((1, 1), (1, 1)))
    full = jnp.zeros_like(m)
    for ky in range(3):
        for kx in range(3):
            full = jnp.maximum(full, mp[ky:ky + h, kx:kx + h])
    rows = lax.broadcasted_iota(jnp.int32, (h // 2, h), 0)
    cols = lax.broadcasted_iota(jnp.int32, (h // 2, h), 1)
    sd = (cols == 2 * rows).astype(jnp.float32)
    t = jnp.dot(full, sd.T, preferred_element_type=jnp.float32)
    return jnp.dot(sd, t, preferred_element_type=jnp.float32)


def _interleave2(x):
    h, wd, c = x.shape
    z = jnp.zeros_like(x)
    r = jnp.concatenate([z[:, None], x[:, None]], axis=1).reshape(2 * h, wd, c)
    return jnp.concatenate([jnp.zeros_like(r)[:, :, None], r[:, :, None]],
                           axis=2).reshape(2 * h, 2 * wd, c)


def _enc1_body(c_ref, w_ref, y_ref, m0_ref, m1_ref, m2_ref, cnt_ref,
               s_ref, q_ref):
    b = pl.program_id(0)
    _P = y_ref.shape[0]
    c1 = s_ref.shape[1]
    cs0 = cs1 = cs2 = jnp.float32(0.0)
    sa = jnp.zeros((c1,), jnp.float32)
    qa = jnp.zeros((c1,), jnp.float32)
    for i in range(_P):
        cnt2 = c_ref[0, i, :, :, 3] + c_ref[1, i, :, :, 3]
        m0 = (cnt2 > 0.0).astype(jnp.float32)
        m1 = _pool_down(m0)
        m2 = _pool_down(m1)
        m0_ref[i] = m0
        m1_ref[i] = m1
        m2_ref[i] = m2
        x = c_ref[0, i, :, :, :3] + c_ref[1, i, :, :, :3]
        y = _conv9(jnp.pad(x, ((1, 1), (1, 1), (0, 0))), w_ref[...])
        y_ref[i] = y
        cs0 += m0.sum()
        cs1 += m1.sum()
        cs2 += m2.sum()
        y2d = y.reshape(HH * WW, c1)
        mflat = m0.reshape(1, HH * WW)
        sa += jnp.dot(mflat, y2d, preferred_element_type=jnp.float32)[0]
        qa += jnp.dot(mflat, y2d * y2d,
                      preferred_element_type=jnp.float32)[0]
    lane = lax.broadcasted_iota(jnp.int32, (1, 128), 1)
    row = (jnp.where(lane == 0, cs0, 0.0)
           + jnp.where(lane == 1, cs1, 0.0)
           + jnp.where(lane == 2, cs2, 0.0))

    @pl.when(b == 0)
    def _():
        cnt_ref[...] = jnp.zeros_like(cnt_ref)
        s_ref[...] = jnp.zeros_like(s_ref)
        q_ref[...] = jnp.zeros_like(q_ref)

    cnt_ref[...] += row
    s_ref[...] += sa[None]
    q_ref[...] += qa[None]


def _enc1(canvas2, w, _P):
    c1 = w.shape[3]
    return pl.pallas_call(
        _enc1_body,
        grid=(BB // _P,),
        in_specs=[
            pl.BlockSpec((2, _P, HH, WW, 4), lambda b: (0, b, 0, 0, 0)),
            pl.BlockSpec((3, 3, 3, c1), lambda b: (0, 0, 0, 0)),
        ],
        out_specs=[
            pl.BlockSpec((_P, HH, WW, c1), lambda b: (b, 0, 0, 0)),
            pl.BlockSpec((_P, HH, WW), lambda b: (b, 0, 0)),
            pl.BlockSpec((_P, HH // 2, WW // 2), lambda b: (b, 0, 0)),
            pl.BlockSpec((_P, HH // 4, WW // 4), lambda b: (b, 0, 0)),
            pl.BlockSpec((1, 128), lambda b: (0, 0)),
            pl.BlockSpec((1, c1), lambda b: (0, 0)),
            pl.BlockSpec((1, c1), lambda b: (0, 0)),
        ],
        out_shape=[
            jax.ShapeDtypeStruct((BB, HH, WW, c1), jnp.float32),
            jax.ShapeDtypeStruct((BB, HH, WW), jnp.float32),
            jax.ShapeDtypeStruct((BB, HH // 2, WW // 2), jnp.float32),
            jax.ShapeDtypeStruct((BB, HH // 4, WW // 4), jnp.float32),
            jax.ShapeDtypeStruct((1, 128), jnp.float32),
            jax.ShapeDtypeStruct((1, c1), jnp.float32),
            jax.ShapeDtypeStruct((1, c1), jnp.float32),
        ],
    )(canvas2, w)


def _layer_body(mode, nin, final,
                x_ref, s_ref, q_ref, cnt_ref, g_ref, b_ref, min_ref,
                w_ref, *rest):
    b = pl.program_id(0)
    if final:
        ob_ref, y_ref = rest
    else:
        mout_ref, y_ref, so_ref, qo_ref = rest
    n = jnp.maximum(cnt_ref[0, nin], 1.0)
    mean = s_ref[0] / n
    var = q_ref[0] / n - mean * mean
    scale = lax.rsqrt(var + EPSV) * g_ref[...]
    shift = b_ref[...] - mean * scale
    _P = x_ref.shape[0]
    if not final:
        cout = so_ref.shape[1]
        sa = jnp.zeros((cout,), jnp.float32)
        qa = jnp.zeros((cout,), jnp.float32)
    for i in range(_P):
        x = x_ref[i]
        h = jnp.maximum(x * scale + shift, 0.0) * min_ref[i][:, :, None]
        if mode == "up":
            xp = jnp.pad(_interleave2(h), ((0, 2), (0, 2), (0, 0)))
        else:
            xp = jnp.pad(h, ((1, 1), (1, 1), (0, 0)))
        y = _conv9(xp, w_ref[...])
        if mode == "down":
            hh = y.shape[0] // 2
            y = y.reshape(hh, 2, hh, 2, y.shape[2])[:, 0, :, 0, :]
        if final:
            y_ref[i] = (y + ob_ref[...]) * min_ref[i][:, :, None]
        else:
            y_ref[i] = y
            ho = y.shape[0]
            y2d = y.reshape(ho * ho, y.shape[2])
            mflat = mout_ref[i].reshape(1, ho * ho)
            sa += jnp.dot(mflat, y2d, preferred_element_type=jnp.float32)[0]
            qa += jnp.dot(mflat, y2d * y2d,
                          preferred_element_type=jnp.float32)[0]
    if final:
        return

    @pl.when(b == 0)
    def _():
        so_ref[...] = jnp.zeros_like(so_ref)
        qo_ref[...] = jnp.zeros_like(qo_ref)

    so_ref[...] += sa[None]
    qo_ref[...] += qa[None]


def _layer(mode, nin, x, stats, cnt, g, bta, m_in, w, m_out=None, ob=None,
           _P=8):
    hin = x.shape[1]
    cin = x.shape[3]
    cout = w.shape[3]
    if mode == "down":
        ho = hin // 2
    elif mode == "up":
        ho = hin * 2
    else:
        ho = hin
    final = ob is not None
    s, q = stats
    in_specs = [
        pl.BlockSpec((_P, hin, hin, cin), lambda b: (b, 0, 0, 0)),
        pl.BlockSpec((1, cin), lambda b: (0, 0)),
        pl.BlockSpec((1, cin), lambda b: (0, 0)),
        pl.BlockSpec((1, 128), lambda b: (0, 0)),
        pl.BlockSpec((cin,), lambda b: (0,)),
        pl.BlockSpec((cin,), lambda b: (0,)),
        pl.BlockSpec((_P, hin, hin), lambda b: (b, 0, 0)),
        pl.BlockSpec((3, 3, cin, cout), lambda b: (0, 0, 0, 0)),
    ]
    args = [x, s, q, cnt, g, bta, m_in, w]
    if final:
        in_specs.append(pl.BlockSpec((cout,), lambda b: (0,)))
        args.append(ob)
        out_specs = [pl.BlockSpec((_P, ho, ho, cout), lambda b: (b, 0, 0, 0))]
        out_shape = [jax.ShapeDtypeStruct((BB, ho, ho, cout), jnp.float32)]
    else:
        in_specs.append(pl.BlockSpec((_P, ho, ho), lambda b: (b, 0, 0)))
        args.append(m_out)
        out_specs = [
            pl.BlockSpec((_P, ho, ho, cout), lambda b: (b, 0, 0, 0)),
            pl.BlockSpec((1, cout), lambda b: (0, 0)),
            pl.BlockSpec((1, cout), lambda b: (0, 0)),
        ]
        out_shape = [
            jax.ShapeDtypeStruct((BB, ho, ho, cout), jnp.float32),
            jax.ShapeDtypeStruct((1, cout), jnp.float32),
            jax.ShapeDtypeStruct((1, cout), jnp.float32),
        ]
    res = pl.pallas_call(
        functools.partial(_layer_body, mode, nin, final),
        grid=(BB // _P,),
        in_specs=in_specs,
        out_specs=out_specs if not final else out_specs[0],
        out_shape=out_shape if not final else out_shape[0],
    )(*args)
    return res


def _forward_tc(canvas2, enc1_W, enc1_g, enc1_b, down1_W, down1_g, down1_b,
                enc2_W, enc2_g, enc2_b, down2_W, down2_g, down2_b,
                up1_W, up1_g, up1_b, dec1_W, dec1_g, dec1_b,
                up2_W, up2_g, up2_b, out_W8, out_b8):
    y1, m0, m1, m2, cnt, s1, q1 = _enc1(canvas2, enc1_W, 1)
    y2, s2, q2 = _layer("down", 0, y1, (s1, q1), cnt, enc1_g, enc1_b, m0,
                        down1_W, m_out=m1, _P=2)
    y3, s3, q3 = _layer("same", 1, y2, (s2, q2), cnt, down1_g, down1_b, m1,
                        enc2_W, m_out=m1, _P=8)
    y4, s4, q4 = _layer("down", 1, y3, (s3, q3), cnt, enc2_g, enc2_b, m1,
                        down2_W, m_out=m2, _P=8)
    y5, s5, q5 = _layer("up", 2, y4, (s4, q4), cnt, down2_g, down2_b, m2,
                        up1_W, m_out=m1, _P=8)
    y6, s6, q6 = _layer("same", 1, y5, (s5, q5), cnt, up1_g, up1_b, m1,
                        dec1_W, m_out=m1, _P=8)
    y7, s7, q7 = _layer("up", 1, y6, (s6, q6), cnt, dec1_g, dec1_b, m1,
                        up2_W, m_out=m0, _P=2)
    rec8 = _layer("same", 0, y7, (s7, q7), cnt, up2_g, up2_b, m0,
                  out_W8, ob=out_b8, _P=2)
    return rec8


def kernel(features, indices, enc1_W, enc1_g, enc1_b, down1_W, down1_g,
           down1_b, enc2_W, enc2_g, enc2_b, down2_W, down2_g, down2_b,
           up1_W, up1_g, up1_b, dec1_W, dec1_g, dec1_b, up2_W, up2_g, up2_b,
           out_W, out_b):
    flat = (indices[:, 0] * HH + indices[:, 1]) * WW + indices[:, 2]
    idx_pad = jnp.zeros((_NPAD,), jnp.int32).at[:NACT].set(flat)
    feat4 = jnp.zeros((_NPAD, 4), jnp.float32)
    feat4 = feat4.at[:NACT, :3].set(features)
    feat4 = feat4.at[:NACT, 3].set(1.0)
    idx3 = idx_pad.reshape(_NW, _NCH, 128)
    idxe = (idx_pad[:, None] * 4 + jnp.arange(4, dtype=jnp.int32)[None])
    idxe = idxe.reshape(_NW, _NCH, 128, 4).transpose(0, 3, 1, 2)
    idxe = idxe.reshape(_NW, 4 * _NCH, 128)
    upd = feat4.reshape(_NW, _NCH, 128, 4).transpose(0, 3, 1, 2)
    upd = upd.reshape(_NW, 4 * _WCHUNK)
    zeros_words = jnp.zeros(((VV * 4) // _NS,), jnp.float32)

    canvas = _sc_scatter(idxe, upd, zeros_words)
    canvas2 = canvas.reshape(_NC, BB, HH, WW, 4)

    out_W16 = jnp.pad(out_W, ((0, 0), (0, 0), (0, 0), (0, 13)))
    out_b16 = jnp.pad(out_b, ((0, 13)))
    rec16 = _forward_tc(canvas2, enc1_W, enc1_g, enc1_b, down1_W, down1_g,
                        down1_b, enc2_W, enc2_g, enc2_b, down2_W, down2_g,
                        down2_b, up1_W, up1_g, up1_b, dec1_W, dec1_g, dec1_b,
                        up2_W, up2_g, up2_b, out_W16, out_b16)

    rows = _sc_gather(rec16.reshape(VV, 16), idx3)
    return rows.reshape(_NPAD, 16)[:NACT, :3]


# speedup vs baseline: 1.2920x; 1.2920x over previous
import functools

import jax
import jax.numpy as jnp
from jax import lax
from jax.experimental import pallas as pl
from jax.experimental.pallas import tpu as pltpu
from jax.experimental.pallas import tpu_sc as plsc

BB, HH, WW = 64, 64, 64
VV = BB * HH * WW
NACT = 50000
EPSV = 1e-5

_NC, _NS = 2, 16
_NW = _NC * _NS
_NCH = 13
_WCHUNK = _NCH * 128
_NPAD = _NW * _WCHUNK
_ZROWS = VV // _NS


def _sc_scatter(idxe, upd, zeros_words):
    mesh = plsc.VectorSubcoreMesh(core_axis_name="c", subcore_axis_name="s")
    zwords = (VV * 4) // _NS

    @functools.partial(
        pl.kernel,
        mesh=mesh,
        compiler_params=pltpu.CompilerParams(use_tc_tiling_on_sc=False),
        out_type=jax.ShapeDtypeStruct((_NC * VV * 4,), jnp.float32),
        scratch_types=[
            pltpu.VMEM((4 * _NCH, 128), jnp.int32),
            pltpu.VMEM((4 * _WCHUNK,), jnp.float32),
            pltpu.VMEM_SHARED((VV * 4,), jnp.float32),
        ],
    )
    def k(idx_hbm, upd_hbm, zero_hbm, out_hbm, idx_v, upd_v, canvas):
        cid = lax.axis_index("c")
        sid = lax.axis_index("s")
        wid = cid * _NS + sid
        w0 = pl.multiple_of(sid * zwords, zwords)
        pltpu.sync_copy(zero_hbm, canvas.at[pl.ds(w0, zwords)])
        pltpu.sync_copy(idx_hbm.at[wid], idx_v)
        pltpu.sync_copy(upd_hbm.at[wid], upd_v)
        plsc.subcore_barrier()
        def body(j, carry):
            off = pl.multiple_of(j * 128, 128)
            pltpu.sync_copy(upd_v.at[pl.ds(off, 128)],
                            canvas.at[idx_v.at[j]], add=True)
            return carry

        lax.fori_loop(0, 4 * _NCH, body, 0)
        plsc.subcore_barrier()
        out0 = pl.multiple_of(cid * VV * 4 + sid * zwords, zwords)
        pltpu.sync_copy(canvas.at[pl.ds(w0, zwords)],
                        out_hbm.at[pl.ds(out0, zwords)])

    return k(idxe, upd, zeros_words)


def _sc_gather(table16, idx3):
    mesh = plsc.VectorSubcoreMesh(core_axis_name="c", subcore_axis_name="s")

    @functools.partial(
        pl.kernel,
        mesh=mesh,
        compiler_params=pltpu.CompilerParams(use_tc_tiling_on_sc=False),
        out_type=jax.ShapeDtypeStruct((_NW, _WCHUNK, 16), jnp.float32),
        scratch_types=[
            pltpu.VMEM((_NCH, 128), jnp.int32),
            pltpu.VMEM((_WCHUNK, 16), jnp.float32),
        ],
    )
    def k(tab_hbm, idx_hbm, out_hbm, idx_v, rows_v):
        cid = lax.axis_index("c")
        sid = lax.axis_index("s")
        wid = cid * _NS + sid
        pltpu.sync_copy(idx_hbm.at[wid], idx_v)
        for j in range(_NCH):
            pltpu.sync_copy(tab_hbm.at[idx_v.at[j]],
                            rows_v.at[pl.ds(j * 128, 128)])
        pltpu.sync_copy(rows_v, out_hbm.at[wid])

    return k(table16, idx3)


def _conv9(xp, w):
    h = xp.shape[0] - 2
    wd = xp.shape[1] - 2
    cin = xp.shape[2]
    cout = w.shape[3]
    acc = jnp.zeros((h * wd, cout), jnp.float32)
    for ky in range(3):
        for kx in range(3):
            patch = xp[ky:ky + h, kx:kx + wd, :].reshape(h * wd, cin)
            acc += jnp.dot(patch, w[ky, kx], preferred_element_type=jnp.float32)
    return acc.reshape(h, wd, cout)


def _pool_down(m):
    h = m.shape[0]
    mp = jnp.pad(m, ((1, 1), (1, 1)))
    full = jnp.zeros_like(m)
    for ky in range(3):
        for kx in range(3):
            full = jnp.maximum(full, mp[ky:ky + h, kx:kx + h])
    return full.reshape(h // 2, 2, h // 2, 2)[:, 0, :, 0]


def _interleave2(x):
    h, wd, c = x.shape
    z = jnp.zeros_like(x)
    r = jnp.concatenate([z[:, None], x[:, None]], axis=1).reshape(2 * h, wd, c)
    return jnp.concatenate([jnp.zeros_like(r)[:, :, None], r[:, :, None]],
                           axis=2).reshape(2 * h, 2 * wd, c)


def _enc1_body(c_ref, w_ref, y_ref, m0_ref, m1_ref, m2_ref, cnt_ref,
               s_ref, q_ref):
    b = pl.program_id(0)
    _P = y_ref.shape[0]
    c1 = s_ref.shape[1]
    cs0 = cs1 = cs2 = jnp.float32(0.0)
    sa = jnp.zeros((c1,), jnp.float32)
    qa = jnp.zeros((c1,), jnp.float32)
    for i in range(_P):
        cnt2 = c_ref[0, i, :, :, 3] + c_ref[1, i, :, :, 3]
        m0 = (cnt2 > 0.0).astype(jnp.float32)
        m1 = _pool_down(m0)
        m2 = _pool_down(m1)
        m0_ref[i] = m0
        m1_ref[i] = m1
        m2_ref[i] = m2
        x = c_ref[0, i, :, :, :3] + c_ref[1, i, :, :, :3]
        y = _conv9(jnp.pad(x, ((1, 1), (1, 1), (0, 0))), w_ref[...])
        y_ref[i] = y
        cs0 += m0.sum()
        cs1 += m1.sum()
        cs2 += m2.sum()
        ym = y * m0[:, :, None]
        sa += ym.sum((0, 1))
        qa += (y * ym).sum((0, 1))
    lane = lax.broadcasted_iota(jnp.int32, (1, 128), 1)
    row = (jnp.where(lane == 0, cs0, 0.0)
           + jnp.where(lane == 1, cs1, 0.0)
           + jnp.where(lane == 2, cs2, 0.0))

    @pl.when(b == 0)
    def _():
        cnt_ref[...] = jnp.zeros_like(cnt_ref)
        s_ref[...] = jnp.zeros_like(s_ref)
        q_ref[...] = jnp.zeros_like(q_ref)

    cnt_ref[...] += row
    s_ref[...] += sa[None]
    q_ref[...] += qa[None]


def _enc1(canvas2, w, _P):
    c1 = w.shape[3]
    return pl.pallas_call(
        _enc1_body,
        grid=(BB // _P,),
        in_specs=[
            pl.BlockSpec((2, _P, HH, WW, 4), lambda b: (0, b, 0, 0, 0)),
            pl.BlockSpec((3, 3, 3, c1), lambda b: (0, 0, 0, 0)),
        ],
        out_specs=[
            pl.BlockSpec((_P, HH, WW, c1), lambda b: (b, 0, 0, 0)),
            pl.BlockSpec((_P, HH, WW), lambda b: (b, 0, 0)),
            pl.BlockSpec((_P, HH // 2, WW // 2), lambda b: (b, 0, 0)),
            pl.BlockSpec((_P, HH // 4, WW // 4), lambda b: (b, 0, 0)),
            pl.BlockSpec((1, 128), lambda b: (0, 0)),
            pl.BlockSpec((1, c1), lambda b: (0, 0)),
            pl.BlockSpec((1, c1), lambda b: (0, 0)),
        ],
        out_shape=[
            jax.ShapeDtypeStruct((BB, HH, WW, c1), jnp.float32),
            jax.ShapeDtypeStruct((BB, HH, WW), jnp.float32),
            jax.ShapeDtypeStruct((BB, HH // 2, WW // 2), jnp.float32),
            jax.ShapeDtypeStruct((BB, HH // 4, WW // 4), jnp.float32),
            jax.ShapeDtypeStruct((1, 128), jnp.float32),
            jax.ShapeDtypeStruct((1, c1), jnp.float32),
            jax.ShapeDtypeStruct((1, c1), jnp.float32),
        ],
    )(canvas2, w)


def _layer_body(mode, nin, final,
                x_ref, s_ref, q_ref, cnt_ref, g_ref, b_ref, min_ref,
                w_ref, *rest):
    b = pl.program_id(0)
    if final:
        ob_ref, y_ref = rest
    else:
        mout_ref, y_ref, so_ref, qo_ref = rest
    n = jnp.maximum(cnt_ref[0, nin], 1.0)
    mean = s_ref[0] / n
    var = q_ref[0] / n - mean * mean
    scale = lax.rsqrt(var + EPSV) * g_ref[...]
    shift = b_ref[...] - mean * scale
    _P = x_ref.shape[0]
    if not final:
        cout = so_ref.shape[1]
        sa = jnp.zeros((cout,), jnp.float32)
        qa = jnp.zeros((cout,), jnp.float32)
    for i in range(_P):
        x = x_ref[i]
        h = jnp.maximum(x * scale + shift, 0.0) * min_ref[i][:, :, None]
        if mode == "up":
            xp = jnp.pad(_interleave2(h), ((0, 2), (0, 2), (0, 0)))
        else:
            xp = jnp.pad(h, ((1, 1), (1, 1), (0, 0)))
        y = _conv9(xp, w_ref[...])
        if mode == "down":
            hh = y.shape[0] // 2
            y = y.reshape(hh, 2, hh, 2, y.shape[2])[:, 0, :, 0, :]
        if final:
            y_ref[i] = (y + ob_ref[...]) * min_ref[i][:, :, None]
        else:
            y_ref[i] = y
            ym = y * mout_ref[i][:, :, None]
            sa += ym.sum((0, 1))
            qa += (y * ym).sum((0, 1))
    if final:
        return

    @pl.when(b == 0)
    def _():
        so_ref[...] = jnp.zeros_like(so_ref)
        qo_ref[...] = jnp.zeros_like(qo_ref)

    so_ref[...] += sa[None]
    qo_ref[...] += qa[None]


def _layer(mode, nin, x, stats, cnt, g, bta, m_in, w, m_out=None, ob=None,
           _P=8):
    hin = x.shape[1]
    cin = x.shape[3]
    cout = w.shape[3]
    if mode == "down":
        ho = hin // 2
    elif mode == "up":
        ho = hin * 2
    else:
        ho = hin
    final = ob is not None
    s, q = stats
    in_specs = [
        pl.BlockSpec((_P, hin, hin, cin), lambda b: (b, 0, 0, 0)),
        pl.BlockSpec((1, cin), lambda b: (0, 0)),
        pl.BlockSpec((1, cin), lambda b: (0, 0)),
        pl.BlockSpec((1, 128), lambda b: (0, 0)),
        pl.BlockSpec((cin,), lambda b: (0,)),
        pl.BlockSpec((cin,), lambda b: (0,)),
        pl.BlockSpec((_P, hin, hin), lambda b: (b, 0, 0)),
        pl.BlockSpec((3, 3, cin, cout), lambda b: (0, 0, 0, 0)),
    ]
    args = [x, s, q, cnt, g, bta, m_in, w]
    if final:
        in_specs.append(pl.BlockSpec((cout,), lambda b: (0,)))
        args.append(ob)
        out_specs = [pl.BlockSpec((_P, ho, ho, cout), lambda b: (b, 0, 0, 0))]
        out_shape = [jax.ShapeDtypeStruct((BB, ho, ho, cout), jnp.float32)]
    else:
        in_specs.append(pl.BlockSpec((_P, ho, ho), lambda b: (b, 0, 0)))
        args.append(m_out)
        out_specs = [
            pl.BlockSpec((_P, ho, ho, cout), lambda b: (b, 0, 0, 0)),
            pl.BlockSpec((1, cout), lambda b: (0, 0)),
            pl.BlockSpec((1, cout), lambda b: (0, 0)),
        ]
        out_shape = [
            jax.ShapeDtypeStruct((BB, ho, ho, cout), jnp.float32),
            jax.ShapeDtypeStruct((1, cout), jnp.float32),
            jax.ShapeDtypeStruct((1, cout), jnp.float32),
        ]
    res = pl.pallas_call(
        functools.partial(_layer_body, mode, nin, final),
        grid=(BB // _P,),
        in_specs=in_specs,
        out_specs=out_specs if not final else out_specs[0],
        out_shape=out_shape if not final else out_shape[0],
    )(*args)
    return res


def _forward_tc(canvas2, enc1_W, enc1_g, enc1_b, down1_W, down1_g, down1_b,
                enc2_W, enc2_g, enc2_b, down2_W, down2_g, down2_b,
                up1_W, up1_g, up1_b, dec1_W, dec1_g, dec1_b,
                up2_W, up2_g, up2_b, out_W8, out_b8):
    y1, m0, m1, m2, cnt, s1, q1 = _enc1(canvas2, enc1_W, 1)
    y2, s2, q2 = _layer("down", 0, y1, (s1, q1), cnt, enc1_g, enc1_b, m0,
                        down1_W, m_out=m1, _P=2)
    y3, s3, q3 = _layer("same", 1, y2, (s2, q2), cnt, down1_g, down1_b, m1,
                        enc2_W, m_out=m1, _P=8)
    y4, s4, q4 = _layer("down", 1, y3, (s3, q3), cnt, enc2_g, enc2_b, m1,
                        down2_W, m_out=m2, _P=8)
    y5, s5, q5 = _layer("up", 2, y4, (s4, q4), cnt, down2_g, down2_b, m2,
                        up1_W, m_out=m1, _P=8)
    y6, s6, q6 = _layer("same", 1, y5, (s5, q5), cnt, up1_g, up1_b, m1,
                        dec1_W, m_out=m1, _P=8)
    y7, s7, q7 = _layer("up", 1, y6, (s6, q6), cnt, dec1_g, dec1_b, m1,
                        up2_W, m_out=m0, _P=2)
    rec8 = _layer("same", 0, y7, (s7, q7), cnt, up2_g, up2_b, m0,
                  out_W8, ob=out_b8, _P=2)
    return rec8


def kernel(features, indices, enc1_W, enc1_g, enc1_b, down1_W, down1_g,
           down1_b, enc2_W, enc2_g, enc2_b, down2_W, down2_g, down2_b,
           up1_W, up1_g, up1_b, dec1_W, dec1_g, dec1_b, up2_W, up2_g, up2_b,
           out_W, out_b):
    flat = (indices[:, 0] * HH + indices[:, 1]) * WW + indices[:, 2]
    idx_pad = jnp.zeros((_NPAD,), jnp.int32).at[:NACT].set(flat)
    feat4 = jnp.zeros((_NPAD, 4), jnp.float32)
    feat4 = feat4.at[:NACT, :3].set(features)
    feat4 = feat4.at[:NACT, 3].set(1.0)
    idx3 = idx_pad.reshape(_NW, _NCH, 128)
    idxe = (idx_pad[:, None] * 4 + jnp.arange(4, dtype=jnp.int32)[None])
    idxe = idxe.reshape(_NW, _NCH, 128, 4).transpose(0, 3, 1, 2)
    idxe = idxe.reshape(_NW, 4 * _NCH, 128)
    upd = feat4.reshape(_NW, _NCH, 128, 4).transpose(0, 3, 1, 2)
    upd = upd.reshape(_NW, 4 * _WCHUNK)
    zeros_words = jnp.zeros(((VV * 4) // _NS,), jnp.float32)

    canvas = _sc_scatter(idxe, upd, zeros_words)
    canvas2 = canvas.reshape(_NC, BB, HH, WW, 4)

    out_W16 = jnp.pad(out_W, ((0, 0), (0, 0), (0, 0), (0, 13)))
    out_b16 = jnp.pad(out_b, ((0, 13)))
    rec16 = _forward_tc(canvas2, enc1_W, enc1_g, enc1_b, down1_W, down1_g,
                        down1_b, enc2_W, enc2_g, enc2_b, down2_W, down2_g,
                        down2_b, up1_W, up1_g, up1_b, dec1_W, dec1_g, dec1_b,
                        up2_W, up2_g, up2_b, out_W16, out_b16)

    rows = _sc_gather(rec16.reshape(VV, 16), idx3)
    return rows.reshape(_NPAD, 16)[:NACT, :3]
